# Initial kernel scaffold; baseline (speedup 1.0000x reference)
import jax
import jax.numpy as jnp
from jax.experimental import pallas as pl


def kernel(x, hex_in, W, b):
    raise NotImplementedError("write your pallas kernel here")


# speedup vs baseline: 2.0978x; 2.0978x over previous
import jax
import jax.numpy as jnp
from jax.experimental import pallas as pl
from jax.experimental.pallas import tpu as pltpu
from jax.experimental.pallas import tpu_sc as plsc

_GATHER_WIN = 128


def _sc_gather(x, idx_flat):
    B = idx_flat.shape[1]
    D = x.shape[1]
    mesh = plsc.VectorSubcoreMesh(core_axis_name="core",
                                  subcore_axis_name="subcore")

    @pl.kernel(out_type=jax.ShapeDtypeStruct((B, D), x.dtype), mesh=mesh)
    def gather_kernel(x_hbm, i_hbm, o_hbm):
        def body(i_vmem, o_vmem):
            pltpu.sync_copy(x_hbm.at[i_vmem.at[0]], o_vmem)

        pltpu.emit_pipeline(
            body,
            grid=(B // _GATHER_WIN,),
            in_specs=[pl.BlockSpec((1, _GATHER_WIN), lambda i: (0, i))],
            out_specs=[pl.BlockSpec((_GATHER_WIN, D), lambda i: (i, 0))],
            core_axis_name=("core", "subcore"),
            dimension_semantics=(pltpu.PARALLEL,),
        )(i_hbm, o_hbm)

    return gather_kernel(x, idx_flat)


def _tc_matmul(g, W, b, block_m, rows):
    K = g.shape[1]
    F = W.shape[1]

    def mm_kernel(g_ref, w_ref, b_ref, o_ref):
        o_ref[...] = (
            jnp.dot(g_ref[...], w_ref[...], preferred_element_type=jnp.float32)
            + b_ref[...]
        )

    return pl.pallas_call(
        mm_kernel,
        grid=(rows // block_m,),
        in_specs=[
            pl.BlockSpec((block_m, K), lambda i: (i, 0)),
            pl.BlockSpec((K, F), lambda i: (0, 0)),
            pl.BlockSpec((1, F), lambda i: (0, 0)),
        ],
        out_specs=pl.BlockSpec((block_m, F), lambda i: (i, 0)),
        out_shape=jax.ShapeDtypeStruct((rows, F), jnp.float32),
    )(g, W, b.reshape(1, F))


def kernel(x, hex_in, W, b):
    N, D = x.shape
    Np = (N + 127) // 128 * 128
    idx = hex_in.astype(jnp.int32).reshape(N * 7)
    idx = jnp.pad(idx, (0, (Np - N) * 7)).reshape(1, Np * 7)
    g = _sc_gather(x, idx)
    g = g.reshape(Np, 7 * D)
    return _tc_matmul(g, W, b, block_m=400, rows=N)


# speedup vs baseline: 3.4771x; 1.6575x over previous
import jax
import jax.numpy as jnp
from jax.experimental import pallas as pl
from jax.experimental.pallas import tpu as pltpu
from jax.experimental.pallas import tpu_sc as plsc

_GATHER_WIN = 128


def _sc_gather_wide(x, idxT):
    K, Np = idxT.shape
    D = x.shape[1]
    mesh = plsc.VectorSubcoreMesh(core_axis_name="core",
                                  subcore_axis_name="subcore")

    @pl.kernel(out_type=jax.ShapeDtypeStruct((Np, K * D), x.dtype), mesh=mesh)
    def gather_kernel(x_hbm, i_hbm, o_hbm):
        def body(i_vmem, o_vmem):
            pltpu.sync_copy(x_hbm.at[i_vmem.at[0]], o_vmem)

        pltpu.emit_pipeline(
            body,
            grid=(K, Np // _GATHER_WIN),
            in_specs=[pl.BlockSpec((1, _GATHER_WIN), lambda k, i: (k, i))],
            out_specs=[pl.BlockSpec((_GATHER_WIN, D), lambda k, i: (i, k))],
            core_axis_name=("core", "subcore"),
            dimension_semantics=(pltpu.PARALLEL, pltpu.PARALLEL),
        )(i_hbm, o_hbm)

    return gather_kernel(x, idxT)


def _tc_matmul(g, W, b, block_m, rows):
    K = g.shape[1]
    F = W.shape[1]

    def mm_kernel(g_ref, w_ref, b_ref, o_ref):
        o_ref[...] = (
            jnp.dot(g_ref[...], w_ref[...], preferred_element_type=jnp.float32)
            + b_ref[...]
        )

    return pl.pallas_call(
        mm_kernel,
        grid=(rows // block_m,),
        in_specs=[
            pl.BlockSpec((block_m, K), lambda i: (i, 0)),
            pl.BlockSpec((K, F), lambda i: (0, 0)),
            pl.BlockSpec((1, F), lambda i: (0, 0)),
        ],
        out_specs=pl.BlockSpec((block_m, F), lambda i: (i, 0)),
        out_shape=jax.ShapeDtypeStruct((rows, F), jnp.float32),
    )(g, W, b.reshape(1, F))


def kernel(x, hex_in, W, b):
    N, D = x.shape
    Np = (N + 127) // 128 * 128
    idxT = jnp.pad(hex_in.astype(jnp.int32).T, ((0, 0), (0, Np - N)))
    g = _sc_gather_wide(x, idxT)
    return _tc_matmul(g, W, b, block_m=400, rows=N)


# speedup vs baseline: 3.6958x; 1.0629x over previous
import jax
import jax.numpy as jnp
from jax.experimental import pallas as pl
from jax.experimental.pallas import tpu as pltpu
from jax.experimental.pallas import tpu_sc as plsc

_GATHER_WIN = 128


def _sc_gather_wide(x, idxT):
    K, Np = idxT.shape
    D = x.shape[1]
    mesh = plsc.VectorSubcoreMesh(core_axis_name="core",
                                  subcore_axis_name="subcore")

    @pl.kernel(out_type=jax.ShapeDtypeStruct((Np, K * D), x.dtype), mesh=mesh)
    def gather_kernel(x_hbm, i_hbm, o_hbm):
        def body(i_vmem, o_vmem):
            pltpu.sync_copy(x_hbm.at[i_vmem.at[0]], o_vmem)

        pltpu.emit_pipeline(
            body,
            grid=(K, Np // _GATHER_WIN),
            in_specs=[pl.BlockSpec((1, _GATHER_WIN), lambda k, i: (k, i))],
            out_specs=[pl.BlockSpec((_GATHER_WIN, D), lambda k, i: (i, k))],
            core_axis_name=("core", "subcore"),
            dimension_semantics=(pltpu.PARALLEL, pltpu.PARALLEL),
        )(i_hbm, o_hbm)

    return gather_kernel(x, idxT)


def _tc_matmul(g, W, b, block_m, rows):
    K = g.shape[1]
    F = W.shape[1]

    def mm_kernel(g_ref, w_ref, b_ref, o_ref):
        o_ref[...] = (
            jnp.dot(g_ref[...], w_ref[...], preferred_element_type=jnp.float32)
            + b_ref[...]
        )

    return pl.pallas_call(
        mm_kernel,
        grid=(rows // block_m,),
        in_specs=[
            pl.BlockSpec((block_m, K), lambda i: (i, 0)),
            pl.BlockSpec((K, F), lambda i: (0, 0)),
            pl.BlockSpec((1, F), lambda i: (0, 0)),
        ],
        out_specs=pl.BlockSpec((block_m, F), lambda i: (i, 0)),
        out_shape=jax.ShapeDtypeStruct((rows, F), jnp.float32),
    )(g, W, b.reshape(1, F))


def kernel(x, hex_in, W, b):
    N, D = x.shape
    Np = (N + 127) // 128 * 128
    idxT = jnp.pad(hex_in.astype(jnp.int32).T, ((0, 0), (0, Np - N)))
    n_chunks = 4
    ch = Np // n_chunks
    outs = []
    for c in range(n_chunks):
        idx_c = jax.lax.slice(idxT, (0, c * ch), (idxT.shape[0], (c + 1) * ch))
        g_c = _sc_gather_wide(x, idx_c)
        outs.append(_tc_matmul(g_c, W, b, block_m=448, rows=ch))
    return jnp.concatenate(outs, axis=0)[:N]


# speedup vs baseline: 3.7664x; 1.0191x over previous
import jax
import jax.numpy as jnp
from jax import lax
from jax.experimental import pallas as pl
from jax.experimental.pallas import tpu as pltpu
from jax.experimental.pallas import tpu_sc as plsc

_WIN = 128
_NW = 32


def _sc_gather_wide(x, idxT):
    K, Npc = idxT.shape
    D = x.shape[1]
    nwin = Npc // _WIN
    J = K * nwin
    Tmax = (J + _NW - 1) // _NW
    mesh = plsc.VectorSubcoreMesh(core_axis_name="core",
                                  subcore_axis_name="subcore")

    @pl.kernel(
        out_type=jax.ShapeDtypeStruct((Npc, K * D), x.dtype),
        mesh=mesh,
        scratch_types=[
            pltpu.VMEM((1, _WIN), jnp.int32),
            pltpu.VMEM((1, _WIN), jnp.int32),
            pltpu.VMEM((_WIN, D), x.dtype),
            pltpu.VMEM((_WIN, D), x.dtype),
            pltpu.SemaphoreType.DMA,
            pltpu.SemaphoreType.DMA,
            pltpu.SemaphoreType.DMA,
            pltpu.SemaphoreType.DMA,
        ],
    )
    def gather_kernel(x_hbm, i_hbm, o_hbm, idx0, idx1, rows0, rows1,
                      g0, g1, w0, w1):
        wid = lax.axis_index("core") * 16 + lax.axis_index("subcore")
        idx_bufs = (idx0, idx1)
        row_bufs = (rows0, rows1)
        gsem = (g0, g1)
        wsem = (w0, w1)

        def load_and_gather(s, j):
            i, k = j // K, j % K
            pltpu.sync_copy(i_hbm.at[pl.ds(k, 1), pl.ds(i * _WIN, _WIN)],
                            idx_bufs[s])
            pltpu.async_copy(x_hbm.at[idx_bufs[s].at[0]], row_bufs[s], gsem[s])

        def gather_wait(s):
            pltpu.make_async_copy(x_hbm.at[idx_bufs[s].at[0]], row_bufs[s],
                                  gsem[s]).wait()

        def _wb_slices(j):
            i, k = j // K, j % K
            return (pl.ds(i * _WIN, _WIN), pl.ds(k * D, D))

        def wb_start(s, j):
            pltpu.async_copy(row_bufs[s], o_hbm.at[_wb_slices(j)], wsem[s])

        def wb_wait(s, j):
            pltpu.make_async_copy(row_bufs[s], o_hbm.at[_wb_slices(j)],
                                  wsem[s]).wait()

        @pl.when(wid < J)
        def _():
            load_and_gather(0, wid)

        @pl.when(wid + _NW < J)
        def _():
            load_and_gather(1, wid + _NW)

        @pl.loop(0, (Tmax + 1) // 2)
        def _(t):
            jA = wid + _NW * (2 * t)
            jB = jA + _NW
            jC = jB + _NW
            jD = jC + _NW

            @pl.when(jA < J)
            def _():
                gather_wait(0)
                wb_start(0, jA)

            @pl.when(jB < J)
            def _():
                gather_wait(1)
                wb_start(1, jB)

            @pl.when(jC < J)
            def _():
                wb_wait(0, jA)
                load_and_gather(0, jC)

            @pl.when(jD < J)
            def _():
                wb_wait(1, jB)
                load_and_gather(1, jD)

        @pl.when(wid < J)
        def _():
            wb_wait(0, 0)

        @pl.when(wid + _NW < J)
        def _():
            wb_wait(1, 0)

    return gather_kernel(x, idxT)


def _tc_matmul(g, W, b, block_m, rows):
    K = g.shape[1]
    F = W.shape[1]

    def mm_kernel(g_ref, w_ref, b_ref, o_ref):
        o_ref[...] = (
            jnp.dot(g_ref[...], w_ref[...], preferred_element_type=jnp.float32)
            + b_ref[...]
        )

    return pl.pallas_call(
        mm_kernel,
        grid=(rows // block_m,),
        in_specs=[
            pl.BlockSpec((block_m, K), lambda i: (i, 0)),
            pl.BlockSpec((K, F), lambda i: (0, 0)),
            pl.BlockSpec((1, F), lambda i: (0, 0)),
        ],
        out_specs=pl.BlockSpec((block_m, F), lambda i: (i, 0)),
        out_shape=jax.ShapeDtypeStruct((rows, F), jnp.float32),
    )(g, W, b.reshape(1, F))


def kernel(x, hex_in, W, b):
    N, D = x.shape
    Np = (N + 127) // 128 * 128
    idxT = jnp.pad(hex_in.astype(jnp.int32).T, ((0, 0), (0, Np - N)))
    n_chunks = 4
    ch = Np // n_chunks
    outs = []
    prev_g = None
    for c in range(n_chunks):
        idx_c = jax.lax.slice(idxT, (0, c * ch), (idxT.shape[0], (c + 1) * ch))
        if prev_g is not None:
            idx_c, _ = jax.lax.optimization_barrier((idx_c, prev_g))
        g_c = _sc_gather_wide(x, idx_c)
        prev_g = g_c
        outs.append(_tc_matmul(g_c, W, b, block_m=448, rows=ch))
    return jnp.concatenate(outs, axis=0)[:N]
